# baseline (speedup 1.0000x reference)
import functools

import jax
import jax.numpy as jnp
from jax import lax
from jax.experimental import pallas as pl
from jax.experimental.pallas import tpu as pltpu
from jax.experimental.pallas import tpu_sc as plsc

B, H, W, C = 8, 224, 224, 96
OUT_H, OUT_W = 224, 224
HW = H * W
N = B * HW
L = 16
NW = 32
PIX_PER_W = N // NW
CK = 32
NCHUNK = PIX_PER_W // CK
RED_CHUNKS = 16
RED_BLK = HW // RED_CHUNKS


def _theta_body(x_ref, w_ref, b_ref, out_ref, acc_ref):
    j = pl.program_id(1)

    @pl.when(j == 0)
    def _():
        acc_ref[...] = jnp.zeros_like(acc_ref)

    acc_ref[...] += jnp.sum(x_ref[0], axis=0, keepdims=True)

    @pl.when(j == RED_CHUNKS - 1)
    def _():
        pooled = acc_ref[...] * (1.0 / HW)
        row = jnp.dot(pooled, w_ref[...],
                      preferred_element_type=jnp.float32) + b_ref[...]
        out_ref[pl.ds(pl.program_id(0), 1), :] = row


def _compute_theta(X, W_loc, b_loc):
    x3 = X.reshape(B, HW, C)
    w_p = jnp.pad(W_loc, ((0, 0), (0, 2)))
    b_p = jnp.pad(b_loc.astype(jnp.float32), (0, 2)).reshape(1, 8)
    return pl.pallas_call(
        _theta_body,
        grid=(B, RED_CHUNKS),
        in_specs=[
            pl.BlockSpec((1, RED_BLK, C), lambda b, j: (b, j, 0)),
            pl.BlockSpec((C, 8), lambda b, j: (0, 0)),
            pl.BlockSpec((1, 8), lambda b, j: (0, 0)),
        ],
        out_specs=pl.BlockSpec((8, 8), lambda b, j: (0, 0)),
        out_shape=jax.ShapeDtypeStruct((B, 8), jnp.float32),
        scratch_shapes=[pltpu.VMEM((1, C), jnp.float32)],
    )(x3, w_p, b_p)


def _bf16_round(x):
    b = lax.bitcast_convert_type(x, jnp.uint32)
    r = (b + 0x7FFF + ((b >> 16) & 1)) & jnp.uint32(0xFFFF0000)
    return lax.bitcast_convert_type(r, jnp.float32)


_mesh = plsc.VectorSubcoreMesh(core_axis_name="c", subcore_axis_name="s")


@functools.partial(
    pl.kernel,
    out_type=jax.ShapeDtypeStruct((N * C,), jnp.float32),
    mesh=_mesh,
    compiler_params=pltpu.CompilerParams(use_tc_tiling_on_sc=False),
    scratch_types=[
        pltpu.VMEM((L,), jnp.float32),
        pltpu.VMEM((4 * CK,), jnp.int32),
        pltpu.VMEM((CK // L, L), jnp.float32),
        pltpu.VMEM((CK // L, L), jnp.float32),
        pltpu.VMEM((CK // L, L), jnp.float32),
        pltpu.VMEM((CK // L, L), jnp.float32),
        pltpu.VMEM((4 * CK, C), jnp.float32),
        pltpu.VMEM((CK * C,), jnp.float32),
        pltpu.SemaphoreType.DMA,
    ],
)
def _sc_sample(x_hbm, theta_hbm, out_hbm,
               th_v, idx_v, wa_v, wb_v, wc_v, wd_v, d_v, o_v, sem):
    wid = lax.axis_index("s") * 2 + lax.axis_index("c")
    batch = wid // 4
    pix0 = wid * PIX_PER_W

    pltpu.sync_copy(theta_hbm.at[batch], th_v)
    th = _bf16_round(th_v[...])
    t00 = th[0]
    t01 = th[1]
    t02 = th[2]
    t10 = th[3]
    t11 = th[4]
    t12 = th[5]

    step = jnp.float32(2.0 / (W - 1))

    def chunk_body(g, _):
        cbase = pix0 + g * CK

        for v in range(CK // L):
            q = (cbase - batch * HW + v * L) + lax.iota(jnp.int32, L)
            py = lax.div(q, jnp.int32(W))
            px = q - py * W
            xn = _bf16_round(px.astype(jnp.float32) * step - 1.0)
            yn = _bf16_round(py.astype(jnp.float32) * step - 1.0)
            xs = t00 * xn + t01 * yn + t02
            ys = t10 * xn + t11 * yn + t12
            xf = 0.5 * (xs + 1.0) * jnp.float32(W)
            yf = 0.5 * (ys + 1.0) * jnp.float32(H)
            x0 = (xf - 0.5).astype(jnp.int32)
            y0 = (yf - 0.5).astype(jnp.int32)
            x1 = x0 + 1
            y1 = y0 + 1
            x0 = jnp.clip(x0, 0, W - 1)
            x1 = jnp.clip(x1, 0, W - 1)
            y0 = jnp.clip(y0, 0, H - 1)
            y1 = jnp.clip(y1, 0, H - 1)
            r0 = batch * HW + y0 * W
            r1 = batch * HW + y1 * W
            sl = pl.ds(v * L, L)
            idx_v[pl.ds(0 * CK + v * L, L)] = r0 + x0
            idx_v[pl.ds(1 * CK + v * L, L)] = r1 + x0
            idx_v[pl.ds(2 * CK + v * L, L)] = r0 + x1
            idx_v[pl.ds(3 * CK + v * L, L)] = r1 + x1
            x0f = x0.astype(jnp.float32)
            x1f = x1.astype(jnp.float32)
            y0f = y0.astype(jnp.float32)
            y1f = y1.astype(jnp.float32)
            wa_v[v, :] = (x1f - xf) * (y1f - yf)
            wb_v[v, :] = (x1f - xf) * (yf - y0f)
            wc_v[v, :] = (xf - x0f) * (y1f - yf)
            wd_v[v, :] = (xf - x0f) * (yf - y0f)

        pltpu.async_copy(x_hbm.at[idx_v], d_v, sem).wait()

        def term_pass(w_ref, soff, first):
            def f(grp, _):
                base = grp * L
                w16 = w_ref[grp, :]
                for j in range(L):
                    row = base + j
                    w = w16[j]
                    for cg in range(C // L):
                        cs = pl.ds(cg * L, L)
                        os = pl.ds(row * C + cg * L, L)
                        t = w * d_v[soff + row, cs]
                        if first:
                            o_v[os] = t
                        else:
                            o_v[os] = o_v[os] + t
                return 0
            lax.fori_loop(0, CK // L, f, 0)

        term_pass(wa_v, 0, True)
        term_pass(wb_v, CK, False)
        term_pass(wc_v, 2 * CK, False)
        term_pass(wd_v, 3 * CK, False)

        pltpu.sync_copy(o_v, out_hbm.at[pl.ds(cbase * C, CK * C)])
        return 0

    lax.fori_loop(0, NCHUNK, chunk_body, 0)


def kernel(X, W_loc, b_loc):
    theta = _compute_theta(X, W_loc, b_loc)
    theta_p = jnp.pad(theta, ((0, 0), (0, L - 8)))
    out = _sc_sample(X.reshape(N, C), theta_p)
    return out.reshape(B, OUT_H, OUT_W, C)


# speedup vs baseline: 1.1431x; 1.1431x over previous
import functools

import jax
import jax.numpy as jnp
from jax import lax
from jax.experimental import pallas as pl
from jax.experimental.pallas import tpu as pltpu
from jax.experimental.pallas import tpu_sc as plsc

B, H, W, C = 8, 224, 224, 96
OUT_H, OUT_W = 224, 224
HW = H * W
N = B * HW
L = 16
NW = 32
PIX_PER_W = N // NW
CK = 32
NCHUNK = PIX_PER_W // CK
RED_CHUNKS = 16
RED_BLK = HW // RED_CHUNKS


def _theta_body(x_ref, w_ref, b_ref, out_ref, acc_ref):
    j = pl.program_id(1)

    @pl.when(j == 0)
    def _():
        acc_ref[...] = jnp.zeros_like(acc_ref)

    acc_ref[...] += jnp.sum(x_ref[0], axis=0, keepdims=True)

    @pl.when(j == RED_CHUNKS - 1)
    def _():
        pooled = acc_ref[...] * (1.0 / HW)
        row = jnp.dot(pooled, w_ref[...],
                      preferred_element_type=jnp.float32) + b_ref[...]
        out_ref[pl.ds(pl.program_id(0), 1), :] = row


def _compute_theta(X, W_loc, b_loc):
    x3 = X.reshape(B, HW, C)
    w_p = jnp.pad(W_loc, ((0, 0), (0, 2)))
    b_p = jnp.pad(b_loc.astype(jnp.float32), (0, 2)).reshape(1, 8)
    return pl.pallas_call(
        _theta_body,
        grid=(B, RED_CHUNKS),
        in_specs=[
            pl.BlockSpec((1, RED_BLK, C), lambda b, j: (b, j, 0)),
            pl.BlockSpec((C, 8), lambda b, j: (0, 0)),
            pl.BlockSpec((1, 8), lambda b, j: (0, 0)),
        ],
        out_specs=pl.BlockSpec((8, 8), lambda b, j: (0, 0)),
        out_shape=jax.ShapeDtypeStruct((B, 8), jnp.float32),
        scratch_shapes=[pltpu.VMEM((1, C), jnp.float32)],
    )(x3, w_p, b_p)


def _bf16_round(x):
    b = lax.bitcast_convert_type(x, jnp.uint32)
    r = (b + 0x7FFF + ((b >> 16) & 1)) & jnp.uint32(0xFFFF0000)
    return lax.bitcast_convert_type(r, jnp.float32)


_mesh = plsc.VectorSubcoreMesh(core_axis_name="c", subcore_axis_name="s")


@functools.partial(
    pl.kernel,
    out_type=jax.ShapeDtypeStruct((N * C,), jnp.float32),
    mesh=_mesh,
    compiler_params=pltpu.CompilerParams(use_tc_tiling_on_sc=False),
    scratch_types=[
        pltpu.VMEM((L,), jnp.float32),
        pltpu.VMEM((4 * CK,), jnp.int32),
        pltpu.VMEM((4 * CK,), jnp.int32),
        pltpu.VMEM((4, CK // L, L), jnp.float32),
        pltpu.VMEM((4, CK // L, L), jnp.float32),
        pltpu.VMEM((4 * CK, C), jnp.float32),
        pltpu.VMEM((4 * CK, C), jnp.float32),
        pltpu.VMEM((CK * C,), jnp.float32),
        pltpu.SemaphoreType.DMA,
        pltpu.SemaphoreType.DMA,
    ],
)
def _sc_sample(x_hbm, theta_hbm, out_hbm,
               th_v, idx0_v, idx1_v, w0_v, w1_v, d0_v, d1_v, o_v, sem0, sem1):
    wid = lax.axis_index("s") * 2 + lax.axis_index("c")
    batch = wid // 4
    pix0 = wid * PIX_PER_W

    pltpu.sync_copy(theta_hbm.at[batch], th_v)
    th = _bf16_round(th_v[...])
    t00 = th[0]
    t01 = th[1]
    t02 = th[2]
    t10 = th[3]
    t11 = th[4]
    t12 = th[5]

    step = jnp.float32(2.0 / (W - 1))

    def compute_chunk(g, idx_v, w_v):
        cbase = pix0 + g * CK
        for v in range(CK // L):
            q = (cbase - batch * HW + v * L) + lax.iota(jnp.int32, L)
            py = lax.div(q, jnp.int32(W))
            px = q - py * W
            xn = _bf16_round(px.astype(jnp.float32) * step - 1.0)
            yn = _bf16_round(py.astype(jnp.float32) * step - 1.0)
            xs = t00 * xn + t01 * yn + t02
            ys = t10 * xn + t11 * yn + t12
            xf = 0.5 * (xs + 1.0) * jnp.float32(W)
            yf = 0.5 * (ys + 1.0) * jnp.float32(H)
            x0 = (xf - 0.5).astype(jnp.int32)
            y0 = (yf - 0.5).astype(jnp.int32)
            x1 = x0 + 1
            y1 = y0 + 1
            x0 = jnp.clip(x0, 0, W - 1)
            x1 = jnp.clip(x1, 0, W - 1)
            y0 = jnp.clip(y0, 0, H - 1)
            y1 = jnp.clip(y1, 0, H - 1)
            r0 = batch * HW + y0 * W
            r1 = batch * HW + y1 * W
            idx_v[pl.ds(0 * CK + v * L, L)] = r0 + x0
            idx_v[pl.ds(1 * CK + v * L, L)] = r1 + x0
            idx_v[pl.ds(2 * CK + v * L, L)] = r0 + x1
            idx_v[pl.ds(3 * CK + v * L, L)] = r1 + x1
            x0f = x0.astype(jnp.float32)
            x1f = x1.astype(jnp.float32)
            y0f = y0.astype(jnp.float32)
            y1f = y1.astype(jnp.float32)
            w_v[0, v, :] = (x1f - xf) * (y1f - yf)
            w_v[1, v, :] = (x1f - xf) * (yf - y0f)
            w_v[2, v, :] = (xf - x0f) * (y1f - yf)
            w_v[3, v, :] = (xf - x0f) * (yf - y0f)

    def blend_and_emit(g, w_v, d_v):
        cbase = pix0 + g * CK

        def term_pass(widx, soff, first):
            def f(grp, _):
                base = grp * L
                w16 = w_v[widx, grp, :]
                for j in range(L):
                    row = base + j
                    w = w16[j]
                    for cg in range(C // L):
                        cs = pl.ds(cg * L, L)
                        os = pl.ds(row * C + cg * L, L)
                        t = w * d_v[soff + row, cs]
                        if first:
                            o_v[os] = t
                        else:
                            o_v[os] = o_v[os] + t
                return 0
            lax.fori_loop(0, CK // L, f, 0)

        term_pass(0, 0, True)
        term_pass(1, CK, False)
        term_pass(2, 2 * CK, False)
        term_pass(3, 3 * CK, False)

        pltpu.sync_copy(o_v, out_hbm.at[pl.ds(cbase * C, CK * C)])

    compute_chunk(0, idx0_v, w0_v)
    pltpu.make_async_copy(x_hbm.at[idx0_v], d0_v, sem0).start()

    def pair_body(m, _):
        g0 = 2 * m
        g1 = g0 + 1
        compute_chunk(g1, idx1_v, w1_v)
        pltpu.make_async_copy(x_hbm.at[idx1_v], d1_v, sem1).start()
        pltpu.make_async_copy(x_hbm.at[idx0_v], d0_v, sem0).wait()
        blend_and_emit(g0, w0_v, d0_v)
        gn = lax.rem(g0 + 2, jnp.int32(NCHUNK))
        compute_chunk(gn, idx0_v, w0_v)
        pltpu.make_async_copy(x_hbm.at[idx0_v], d0_v, sem0).start()
        pltpu.make_async_copy(x_hbm.at[idx1_v], d1_v, sem1).wait()
        blend_and_emit(g1, w1_v, d1_v)
        return 0

    lax.fori_loop(0, NCHUNK // 2, pair_body, 0)
    pltpu.make_async_copy(x_hbm.at[idx0_v], d0_v, sem0).wait()


def kernel(X, W_loc, b_loc):
    theta = _compute_theta(X, W_loc, b_loc)
    theta_p = jnp.pad(theta, ((0, 0), (0, L - 8)))
    out = _sc_sample(X.reshape(N, C), theta_p)
    return out.reshape(B, OUT_H, OUT_W, C)


# speedup vs baseline: 1.2646x; 1.1063x over previous
import functools

import jax
import jax.numpy as jnp
from jax import lax
from jax.experimental import pallas as pl
from jax.experimental.pallas import tpu as pltpu
from jax.experimental.pallas import tpu_sc as plsc

B, H, W, C = 8, 224, 224, 96
OUT_H, OUT_W = 224, 224
HW = H * W
N = B * HW
L = 16
NW = 32
PIX_PER_W = N // NW
CK = 32
NCHUNK = PIX_PER_W // CK
RED_CHUNKS = 16
RED_BLK = HW // RED_CHUNKS


def _theta_body(x_ref, w_ref, b_ref, out_ref, acc_ref):
    j = pl.program_id(1)

    @pl.when(j == 0)
    def _():
        acc_ref[...] = jnp.zeros_like(acc_ref)

    acc_ref[...] += jnp.sum(x_ref[0], axis=0, keepdims=True)

    @pl.when(j == RED_CHUNKS - 1)
    def _():
        pooled = acc_ref[...] * (1.0 / HW)
        row = jnp.dot(pooled, w_ref[...],
                      preferred_element_type=jnp.float32) + b_ref[...]
        out_ref[pl.ds(pl.program_id(0), 1), :] = row


def _compute_theta(X, W_loc, b_loc):
    x3 = X.reshape(B, HW, C)
    w_p = jnp.pad(W_loc, ((0, 0), (0, 2)))
    b_p = jnp.pad(b_loc.astype(jnp.float32), (0, 2)).reshape(1, 8)
    return pl.pallas_call(
        _theta_body,
        grid=(B, RED_CHUNKS),
        in_specs=[
            pl.BlockSpec((1, RED_BLK, C), lambda b, j: (b, j, 0)),
            pl.BlockSpec((C, 8), lambda b, j: (0, 0)),
            pl.BlockSpec((1, 8), lambda b, j: (0, 0)),
        ],
        out_specs=pl.BlockSpec((8, 8), lambda b, j: (0, 0)),
        out_shape=jax.ShapeDtypeStruct((B, 8), jnp.float32),
        scratch_shapes=[pltpu.VMEM((1, C), jnp.float32)],
    )(x3, w_p, b_p)


def _bf16_round(x):
    b = lax.bitcast_convert_type(x, jnp.uint32)
    r = (b + 0x7FFF + ((b >> 16) & 1)) & jnp.uint32(0xFFFF0000)
    return lax.bitcast_convert_type(r, jnp.float32)


_mesh = plsc.VectorSubcoreMesh(core_axis_name="c", subcore_axis_name="s")


@functools.partial(
    pl.kernel,
    out_type=jax.ShapeDtypeStruct((N * C,), jnp.float32),
    mesh=_mesh,
    compiler_params=pltpu.CompilerParams(use_tc_tiling_on_sc=False),
    scratch_types=[
        pltpu.VMEM((L,), jnp.float32),
        pltpu.VMEM((4 * CK,), jnp.int32),
        pltpu.VMEM((4 * CK,), jnp.int32),
        pltpu.VMEM((4, CK // L, L), jnp.float32),
        pltpu.VMEM((4, CK // L, L), jnp.float32),
        pltpu.VMEM((4 * CK, C), jnp.float32),
        pltpu.VMEM((4 * CK, C), jnp.float32),
        pltpu.VMEM((CK * C,), jnp.float32),
        pltpu.SemaphoreType.DMA,
        pltpu.SemaphoreType.DMA,
    ],
)
def _sc_sample(x_hbm, theta_hbm, out_hbm,
               th_v, idx0_v, idx1_v, w0_v, w1_v, d0_v, d1_v, o_v, sem0, sem1):
    wid = lax.axis_index("s") * 2 + lax.axis_index("c")
    batch = wid // 4
    pix0 = wid * PIX_PER_W

    pltpu.sync_copy(theta_hbm.at[batch], th_v)
    th = _bf16_round(th_v[...])
    t00 = th[0]
    t01 = th[1]
    t02 = th[2]
    t10 = th[3]
    t11 = th[4]
    t12 = th[5]

    step = jnp.float32(2.0 / (W - 1))

    def compute_chunk(g, idx_v, w_v):
        cbase = pix0 + g * CK
        for v in range(CK // L):
            q = (cbase - batch * HW + v * L) + lax.iota(jnp.int32, L)
            py = lax.div(q, jnp.int32(W))
            px = q - py * W
            xn = _bf16_round(px.astype(jnp.float32) * step - 1.0)
            yn = _bf16_round(py.astype(jnp.float32) * step - 1.0)
            xs = t00 * xn + t01 * yn + t02
            ys = t10 * xn + t11 * yn + t12
            xf = 0.5 * (xs + 1.0) * jnp.float32(W)
            yf = 0.5 * (ys + 1.0) * jnp.float32(H)
            x0 = (xf - 0.5).astype(jnp.int32)
            y0 = (yf - 0.5).astype(jnp.int32)
            x1 = x0 + 1
            y1 = y0 + 1
            x0 = jnp.clip(x0, 0, W - 1)
            x1 = jnp.clip(x1, 0, W - 1)
            y0 = jnp.clip(y0, 0, H - 1)
            y1 = jnp.clip(y1, 0, H - 1)
            r0 = batch * HW + y0 * W
            r1 = batch * HW + y1 * W
            idx_v[pl.ds(0 * CK + v * L, L)] = r0 + x0
            idx_v[pl.ds(1 * CK + v * L, L)] = r1 + x0
            idx_v[pl.ds(2 * CK + v * L, L)] = r0 + x1
            idx_v[pl.ds(3 * CK + v * L, L)] = r1 + x1
            x0f = x0.astype(jnp.float32)
            x1f = x1.astype(jnp.float32)
            y0f = y0.astype(jnp.float32)
            y1f = y1.astype(jnp.float32)
            w_v[0, v, :] = (x1f - xf) * (y1f - yf)
            w_v[1, v, :] = (x1f - xf) * (yf - y0f)
            w_v[2, v, :] = (xf - x0f) * (y1f - yf)
            w_v[3, v, :] = (xf - x0f) * (yf - y0f)

    def blend_and_emit(g, w_v, d_v):
        cbase = pix0 + g * CK

        def blend_grp(grp, _):
            base = grp * L
            wa16 = w_v[0, grp, :]
            wb16 = w_v[1, grp, :]
            wc16 = w_v[2, grp, :]
            wd16 = w_v[3, grp, :]
            for j in range(L):
                row = base + j
                wa = wa16[j]
                wb = wb16[j]
                wc = wc16[j]
                wd = wd16[j]
                for cg in range(C // L):
                    cs = pl.ds(cg * L, L)
                    os = pl.ds(row * C + cg * L, L)
                    o_v[os] = (wa * d_v[row, cs] + wb * d_v[CK + row, cs] +
                               wc * d_v[2 * CK + row, cs] +
                               wd * d_v[3 * CK + row, cs])
            return 0

        lax.fori_loop(0, CK // L, blend_grp, 0)

        pltpu.sync_copy(o_v, out_hbm.at[pl.ds(cbase * C, CK * C)])

    compute_chunk(0, idx0_v, w0_v)
    pltpu.make_async_copy(x_hbm.at[idx0_v], d0_v, sem0).start()

    def pair_body(m, _):
        g0 = 2 * m
        g1 = g0 + 1
        compute_chunk(g1, idx1_v, w1_v)
        pltpu.make_async_copy(x_hbm.at[idx1_v], d1_v, sem1).start()
        pltpu.make_async_copy(x_hbm.at[idx0_v], d0_v, sem0).wait()
        blend_and_emit(g0, w0_v, d0_v)
        gn = lax.rem(g0 + 2, jnp.int32(NCHUNK))
        compute_chunk(gn, idx0_v, w0_v)
        pltpu.make_async_copy(x_hbm.at[idx0_v], d0_v, sem0).start()
        pltpu.make_async_copy(x_hbm.at[idx1_v], d1_v, sem1).wait()
        blend_and_emit(g1, w1_v, d1_v)
        return 0

    lax.fori_loop(0, NCHUNK // 2, pair_body, 0)
    pltpu.make_async_copy(x_hbm.at[idx0_v], d0_v, sem0).wait()


def kernel(X, W_loc, b_loc):
    theta = _compute_theta(X, W_loc, b_loc)
    theta_p = jnp.pad(theta, ((0, 0), (0, L - 8)))
    out = _sc_sample(X.reshape(N, C), theta_p)
    return out.reshape(B, OUT_H, OUT_W, C)
